# baseline (speedup 1.0000x reference)
import functools

import jax
import jax.numpy as jnp
from jax import lax
from jax.experimental import pallas as pl
from jax.experimental.pallas import tpu as pltpu
from jax.experimental.pallas import tpu_sc as plsc

NC = 2
NS = 16
NW = NC * NS
SB = 64


@functools.partial(jax.jit, static_argnums=(1,))
def _extract_lastcol(table, blk):
    v, d = table.shape
    lc_rows = (v + 127) // 128
    n_blocks = (v + blk - 1) // blk
    full_blocks = v // blk
    tail_rows = v - full_blocks * blk
    blocks_per_w = (n_blocks + NW - 1) // NW
    sub = blk // SB
    mesh = plsc.VectorSubcoreMesh(core_axis_name="c", subcore_axis_name="s")

    @functools.partial(
        pl.kernel,
        mesh=mesh,
        out_type=jax.ShapeDtypeStruct((lc_rows, 128), jnp.float32),
        compiler_params=pltpu.CompilerParams(needs_layout_passes=False),
        scratch_types=[
            [pltpu.VMEM((SB, 1), jnp.float32) for _ in range(8)],
            pltpu.VMEM((blk // 128, 128), jnp.float32),
            pltpu.SemaphoreType.DMA,
            pltpu.SemaphoreType.DMA,
        ],
    )
    def ka(table_hbm, lc_hbm, tmps, acc_v, sem, osem):
        cid = lax.axis_index("c")
        sid = lax.axis_index("s")
        wid = sid * NC + cid

        zeros16 = jnp.zeros((16,), jnp.int32)
        iota16 = lax.iota(jnp.int32, 16)

        def do_block(b, n_sub):
            for s0 in range(0, n_sub, 8):
                grp = min(8, n_sub - s0)
                for i in range(grp):
                    s = s0 + i
                    pltpu.async_copy(
                        table_hbm.at[
                            pl.ds(b * blk + s * SB, SB), pl.ds(d - 1, 1)
                        ],
                        tmps[i],
                        sem,
                    )
                for i in range(grp):
                    pltpu.make_async_copy(
                        table_hbm.at[pl.ds(0, SB), pl.ds(d - 1, 1)],
                        tmps[i],
                        sem,
                    ).wait()
                for i in range(grp):
                    s = s0 + i
                    for l in range(SB // 16):
                        p = s * SB + l * 16
                        vals = plsc.load_gather(
                            tmps[i], [iota16 + l * 16, zeros16]
                        )
                        acc_v[p // 128, pl.ds(p % 128, 16)] = vals

        def blk_body(g, carry):
            b = wid + NW * g

            @pl.when(b < full_blocks)
            def _full():
                do_block(b, sub)
                pltpu.async_copy(
                    acc_v, lc_hbm.at[pl.ds(b * (blk // 128), blk // 128)],
                    osem,
                ).wait()

            @pl.when(b == full_blocks)
            def _tail():
                do_block(b, tail_rows // SB)
                pltpu.async_copy(
                    acc_v.at[pl.ds(0, 4)],
                    lc_hbm.at[pl.ds(full_blocks * (blk // 128), 4)],
                    osem,
                ).wait()
                pltpu.async_copy(
                    acc_v.at[pl.ds(4, 1), pl.ds(0, SB)],
                    lc_hbm.at[pl.ds(lc_rows - 1, 1), pl.ds(0, SB)],
                    osem,
                ).wait()

            return carry

        lax.fori_loop(0, blocks_per_w, blk_body, 0, unroll=False)

    return ka(table)


@functools.partial(jax.jit, static_argnums=(3,))
def _lookup(flat_idx, table, lastcol, chunk):
    n = flat_idx.shape[0]
    v, d = table.shape
    lc_rows = lastcol.shape[0]
    n_per_w = n // NW
    n_chunks = n_per_w // chunk
    lc_per_s = 488
    mesh = plsc.VectorSubcoreMesh(core_axis_name="c", subcore_axis_name="s")

    @functools.partial(
        pl.kernel,
        mesh=mesh,
        out_type=jax.ShapeDtypeStruct((n, d), jnp.float32),
        compiler_params=pltpu.CompilerParams(needs_layout_passes=False),
        scratch_types=[
            pltpu.VMEM((chunk,), jnp.int32),
            pltpu.VMEM((chunk,), jnp.int32),
            pltpu.VMEM((chunk, d), jnp.float32),
            pltpu.VMEM((chunk, 128), jnp.float32),
            pltpu.VMEM_SHARED((lc_rows, 128), jnp.float32),
            pltpu.SemaphoreType.DMA,
            pltpu.SemaphoreType.DMA,
        ],
    )
    def k(idx_hbm, table_hbm, lc_hbm, out_hbm, idx_v, idxhi_v, rows_v,
          slab_v, lc_sh, sem, sem2):
        cid = lax.axis_index("c")
        sid = lax.axis_index("s")
        wid = sid * NC + cid
        base = wid * n_per_w

        so = sid * lc_per_s
        pltpu.sync_copy(
            lc_hbm.at[pl.ds(so, lc_per_s)], lc_sh.at[pl.ds(so, lc_per_s)]
        )

        @pl.when(sid == NS - 1)
        def _stage_tail():
            pltpu.sync_copy(
                lc_hbm.at[pl.ds(NS * lc_per_s, lc_rows - NS * lc_per_s)],
                lc_sh.at[pl.ds(NS * lc_per_s, lc_rows - NS * lc_per_s)],
            )

        plsc.subcore_barrier()

        def chunk_body(c, carry):
            off = base + c * chunk
            pltpu.sync_copy(idx_hbm.at[pl.ds(off, chunk)], idx_v)
            for g in range(chunk // 16):
                iv = idx_v[pl.ds(g * 16, 16)]
                idxhi_v[pl.ds(g * 16, 16)] = lax.shift_right_logical(iv, 7)
            main = pltpu.async_copy(
                table_hbm.at[idx_v, pl.ds(0, d - 1)],
                rows_v.at[:, pl.ds(0, d - 1)],
                sem,
            )
            pltpu.async_copy(lc_sh.at[idxhi_v], slab_v, sem2).wait()
            main.wait()
            for g in range(chunk // 16):
                iv = idx_v[pl.ds(g * 16, 16)]
                lo = lax.bitwise_and(iv, 127)
                rows16 = lax.iota(jnp.int32, 16) + g * 16
                vals = plsc.load_gather(slab_v, [rows16, lo])
                plsc.store_scatter(
                    rows_v, [rows16, jnp.full((16,), d - 1, jnp.int32)], vals
                )
            pltpu.sync_copy(rows_v, out_hbm.at[pl.ds(off, chunk)])
            return carry

        lax.fori_loop(0, n_chunks, chunk_body, 0, unroll=False)

    return k(flat_idx, table, lastcol)


def kernel(indices, embeddings):
    b, s = indices.shape
    d = embeddings.shape[1]
    flat_idx = indices.reshape(b * s).astype(jnp.int32)
    lastcol = _extract_lastcol(embeddings, 1024)
    out = _lookup(flat_idx, embeddings, lastcol, 128)
    return out.reshape(b, s, d)


# speedup vs baseline: 1.2549x; 1.2549x over previous
import functools

import jax
import jax.numpy as jnp
from jax import lax
from jax.experimental import pallas as pl
from jax.experimental.pallas import tpu as pltpu
from jax.experimental.pallas import tpu_sc as plsc

NC = 2
NS = 16
NW = NC * NS
LC_ROWS = 7816
OC = 4


@jax.jit
def _lookup(idx2, table, lastcol):
    b, s = idx2.shape
    v, d = table.shape
    rows_per_w = b // NW
    chunk = OC * s
    n_steps = rows_per_w // OC
    lc_per_s = 488
    ngrp = chunk // 16
    mesh = plsc.VectorSubcoreMesh(core_axis_name="c", subcore_axis_name="s")

    @functools.partial(
        pl.kernel,
        mesh=mesh,
        out_type=jax.ShapeDtypeStruct((b, s, d), jnp.float32),
        compiler_params=pltpu.CompilerParams(needs_layout_passes=False),
        scratch_types=[
            pltpu.VMEM((2 * OC, s), jnp.int32),
            pltpu.VMEM((chunk,), jnp.int32),
            pltpu.VMEM((OC, s, d), jnp.float32),
            pltpu.VMEM((chunk, 128), jnp.float32),
            pltpu.VMEM_SHARED((LC_ROWS, 128), jnp.float32),
            pltpu.SemaphoreType.DMA,
            pltpu.SemaphoreType.DMA,
        ],
    )
    def k(idx_hbm, table_hbm, lc_hbm, out_hbm, idx2_v, idxhi_v,
          rows_v, slab_v, lc_sh, sem, sem2):
        cid = lax.axis_index("c")
        sid = lax.axis_index("s")
        wid = sid * NC + cid
        row_base = wid * rows_per_w
        iota16 = lax.iota(jnp.int32, 16)
        p_outer = [
            lax.div(iota16 + g * 16, jnp.int32(s)) for g in range(ngrp)
        ]
        p_inner = [
            lax.rem(iota16 + g * 16, jnp.int32(s)) for g in range(ngrp)
        ]
        col_last = jnp.full((16,), d - 1, jnp.int32)

        so = sid * lc_per_s
        pltpu.sync_copy(
            lc_hbm.at[pl.ds(so, lc_per_s)], lc_sh.at[pl.ds(so, lc_per_s)]
        )

        @pl.when(sid == NS - 1)
        def _stage_tail():
            pltpu.sync_copy(
                lc_hbm.at[pl.ds(NS * lc_per_s, LC_ROWS - NS * lc_per_s)],
                lc_sh.at[pl.ds(NS * lc_per_s, LC_ROWS - NS * lc_per_s)],
            )

        plsc.subcore_barrier()

        def step_body(c, carry):
            outer = row_base + c * OC
            par = lax.rem(c, 2) * OC

            @pl.when(lax.rem(c, 2) == 0)
            def _refresh():
                offr = pl.multiple_of(
                    row_base + lax.div(c, 2) * (2 * OC), 2 * OC
                )
                pltpu.sync_copy(idx_hbm.at[pl.ds(offr, 2 * OC)], idx2_v)

            for g in range(ngrp):
                iv = plsc.load_gather(
                    idx2_v, [p_outer[g] + par, p_inner[g]]
                )
                idxhi_v[pl.ds(g * 16, 16)] = lax.shift_right_logical(iv, 7)
            mains = []
            for j in range(OC):
                mains.append(
                    pltpu.async_copy(
                        table_hbm.at[idx2_v.at[par + j], pl.ds(0, d - 1)],
                        rows_v.at[j, :, pl.ds(0, d - 1)],
                        sem,
                    )
                )
            pltpu.async_copy(lc_sh.at[idxhi_v], slab_v, sem2).wait()
            for m in mains:
                m.wait()
            for g in range(ngrp):
                iv = plsc.load_gather(
                    idx2_v, [p_outer[g] + par, p_inner[g]]
                )
                lo = lax.bitwise_and(iv, 127)
                rows16 = iota16 + g * 16
                vals = plsc.load_gather(slab_v, [rows16, lo])
                plsc.store_scatter(
                    rows_v, [p_outer[g], p_inner[g], col_last], vals
                )
            for j in range(OC):
                pltpu.sync_copy(rows_v.at[j], out_hbm.at[outer + j])
            return carry

        lax.fori_loop(0, n_steps, step_body, 0, unroll=False)

    return k(idx2, table, lastcol)


def kernel(indices, embeddings):
    b, s = indices.shape
    v, d = embeddings.shape
    idx2 = indices.astype(jnp.int32)
    lastcol = jnp.pad(
        embeddings[:, d - 1], (0, LC_ROWS * 128 - v)
    ).reshape(LC_ROWS, 128)
    return _lookup(idx2, embeddings, lastcol)
